# baseline (speedup 1.0000x reference)
import functools
import math

import jax
import jax.numpy as jnp
from jax import lax
from jax.experimental import pallas as pl
from jax.experimental.pallas import tpu as pltpu
from jax.experimental.pallas import tpu_sc as plsc

B = 1024
C = 100000
SCALE = 64.0

_NC = 2
_NS = 16
_NW = _NC * _NS
_BPW = B // _NW
_L = 16

_W = 1536
_NCHUNK = 64
_CSPLIT = _W * _NCHUNK
_TAIL_BN = 2048


def _sc_gather_body(logits_hbm, labels_hbm, out_hbm, lab_v, win_v, out_v, sem):
    wid = lax.axis_index("s") * _NC + lax.axis_index("c")
    base = wid * _BPW
    pltpu.sync_copy(labels_hbm.at[pl.ds(base, _BPW)], lab_v)
    copies = []
    for r in range(_BPW):
        chunk = jnp.maximum(lab_v[pl.ds((r // _L) * _L, _L)], 0)
        lab_r = chunk[r % _L]
        col128 = pl.multiple_of(lab_r & -128, 128)
        row8 = pl.multiple_of(base + (r & -8), 8)
        copies.append(
            pltpu.make_async_copy(
                logits_hbm.at[pl.ds(row8, 8), pl.ds(col128, 128)],
                win_v.at[pl.ds(r * 8, 8), :],
                sem,
            )
        )
        copies[-1].start()
    for cp in copies:
        cp.wait()
    for ch in range(_BPW // _L):
        labs = jnp.maximum(lab_v[pl.ds(ch * _L, _L)], 0)
        ridx = lax.iota(jnp.int32, _L) + ch * _L
        out_v[pl.ds(ch * _L, _L)] = plsc.load_gather(
            win_v, [ridx * 8 + (ridx & 7), labs & 127]
        )
    pltpu.sync_copy(out_v, out_hbm.at[pl.ds(base, _BPW)])


@functools.cache
def _sc_gather():
    return pl.kernel(
        _sc_gather_body,
        out_type=jax.ShapeDtypeStruct((B,), jnp.float32),
        mesh=plsc.VectorSubcoreMesh(core_axis_name="c", subcore_axis_name="s"),
        scratch_types=[
            pltpu.VMEM((_BPW,), jnp.int32),
            pltpu.VMEM((_BPW * 8, 128), jnp.float32),
            pltpu.VMEM((_BPW,), jnp.float32),
            pltpu.SemaphoreType.DMA,
        ],
        compiler_params=pltpu.CompilerParams(needs_layout_passes=False),
    )


def _tc_nv_body(margin_ref, target_ref, nv_ref):
    m = margin_ref[...]
    t = target_ref[...]
    cos_m = jnp.cos(m)
    sin_m = jnp.sin(m)
    theta = jnp.cos(math.pi - m)
    sinmm = jnp.sin(math.pi - m) * m
    sin_t = jnp.sqrt(1.0 - t * t)
    cos_tm = t * cos_m - sin_t * sin_m
    nv_ref[...] = jnp.where(t > theta, cos_tm, t - sinmm) * SCALE


def _tc_nv(margin, target):
    return pl.pallas_call(
        _tc_nv_body,
        out_shape=jax.ShapeDtypeStruct((B, 1), jnp.float32),
    )(margin, target)


def _sc_stream_body(
    x_hbm, labels_hbm, nv_hbm, o_hbm, lab_v, nv_v, buf0, buf1, sin0, sin1, so0, so1
):
    wid = lax.axis_index("s") * _NC + lax.axis_index("c")
    base = pl.multiple_of(wid * _BPW, _BPW)
    pltpu.sync_copy(labels_hbm.at[pl.ds(base, _BPW)], lab_v)
    pltpu.sync_copy(nv_hbm.at[pl.ds(base, _BPW)], nv_v)
    bufs = (buf0, buf1)
    sins = (sin0, sin1)
    souts = (so0, so1)

    @pl.loop(0, _NCHUNK, step=2)
    def _chunks(g):
        for b in range(2):
            buf = bufs[b]
            c0 = pl.multiple_of((g + b) * _W, 128)
            cin = pltpu.make_async_copy(
                x_hbm.at[pl.ds(base, _BPW), pl.ds(c0, _W)], buf, sins[b]
            )
            cin.start()
            cin.wait()

            @pl.loop(0, _W // _L)
            def _scale(i):
                col = i * _L
                for r in range(_BPW):
                    buf[r, pl.ds(col, _L)] = buf[r, pl.ds(col, _L)] * SCALE

            for ch in range(_BPW // _L):
                labs = lab_v[pl.ds(ch * _L, _L)]
                nvs = nv_v[pl.ds(ch * _L, _L)]
                ridx = lax.iota(jnp.int32, _L) + ch * _L
                mask = (labs >= c0) & (labs < c0 + _W)
                plsc.store_scatter(buf, [ridx, labs - c0], nvs, mask=mask)

            cout = pltpu.make_async_copy(
                buf, o_hbm.at[pl.ds(base, _BPW), pl.ds(c0, _W)], souts[b]
            )
            cout.start()
            cout.wait()


@functools.cache
def _sc_stream():
    return pl.kernel(
        _sc_stream_body,
        out_type=jax.ShapeDtypeStruct((B, C), jnp.float32),
        mesh=plsc.VectorSubcoreMesh(core_axis_name="c", subcore_axis_name="s"),
        scratch_types=[
            pltpu.VMEM((_BPW,), jnp.int32),
            pltpu.VMEM((_BPW,), jnp.float32),
            pltpu.VMEM((_BPW, _W), jnp.float32),
            pltpu.VMEM((_BPW, _W), jnp.float32),
            pltpu.SemaphoreType.DMA,
            pltpu.SemaphoreType.DMA,
            pltpu.SemaphoreType.DMA,
            pltpu.SemaphoreType.DMA,
        ],
        compiler_params=pltpu.CompilerParams(needs_layout_passes=False),
    )


def _tc_tail_body(o_ref, labels_ref, nv_ref, logits_ref, out_ref):
    x = logits_ref[...]
    rel = labels_ref[...] - _CSPLIT
    cols = lax.broadcasted_iota(jnp.int32, (B, _TAIL_BN), 1)
    out_ref[...] = jnp.where(cols == rel, nv_ref[...], x * SCALE)


def _tc_tail(o1, labels, nv, logits):
    tail_spec = pl.BlockSpec((B, _TAIL_BN), lambda i: (0, _CSPLIT // _TAIL_BN))
    row_spec = pl.BlockSpec((B, 1), lambda i: (0, 0))
    return pl.pallas_call(
        _tc_tail_body,
        grid=(1,),
        in_specs=[
            pl.BlockSpec((8, 128), lambda i: (0, 0)),
            row_spec,
            row_spec,
            tail_spec,
        ],
        out_specs=tail_spec,
        out_shape=jax.ShapeDtypeStruct((B, C), jnp.float32),
        input_output_aliases={0: 0},
    )(o1, labels, nv, logits)


def kernel(margin, logits, labels):
    labels32 = labels.astype(jnp.int32)
    target = _sc_gather()(logits, labels32)
    nv = _tc_nv(margin.reshape(B, 1), target.reshape(B, 1))
    o1 = _sc_stream()(logits, labels32, nv.reshape(B))
    return _tc_tail(o1, labels32.reshape(B, 1), nv, logits)


# speedup vs baseline: 1.2366x; 1.2366x over previous
import functools
import math

import jax
import jax.numpy as jnp
from jax import lax
from jax.experimental import pallas as pl
from jax.experimental.pallas import tpu as pltpu
from jax.experimental.pallas import tpu_sc as plsc

B = 1024
C = 100000
SCALE = 64.0


_NC = 2
_NS = 16
_NW = _NC * _NS
_BPW = B // _NW
_L = 16


def _sc_gather_body(logits_hbm, labels_hbm, out_hbm, lab_v, win_v, out_v, sem):
    wid = lax.axis_index("s") * _NC + lax.axis_index("c")
    base = wid * _BPW
    pltpu.sync_copy(labels_hbm.at[pl.ds(base, _BPW)], lab_v)
    copies = []
    for r in range(_BPW):
        chunk = jnp.maximum(lab_v[pl.ds((r // _L) * _L, _L)], 0)
        lab_r = chunk[r % _L]
        col128 = pl.multiple_of(lab_r & -128, 128)
        row8 = pl.multiple_of(base + (r & -8), 8)
        copies.append(
            pltpu.make_async_copy(
                logits_hbm.at[pl.ds(row8, 8), pl.ds(col128, 128)],
                win_v.at[pl.ds(r * 8, 8), :],
                sem,
            )
        )
        copies[-1].start()
    for cp in copies:
        cp.wait()
    for ch in range(_BPW // _L):
        labs = jnp.maximum(lab_v[pl.ds(ch * _L, _L)], 0)
        ridx = lax.iota(jnp.int32, _L) + ch * _L
        out_v[pl.ds(ch * _L, _L)] = plsc.load_gather(
            win_v, [ridx * 8 + (ridx & 7), labs & 127]
        )
    pltpu.sync_copy(out_v, out_hbm.at[pl.ds(base, _BPW)])


@functools.cache
def _sc_gather():
    return pl.kernel(
        _sc_gather_body,
        out_type=jax.ShapeDtypeStruct((B,), jnp.float32),
        mesh=plsc.VectorSubcoreMesh(core_axis_name="c", subcore_axis_name="s"),
        scratch_types=[
            pltpu.VMEM((_BPW,), jnp.int32),
            pltpu.VMEM((_BPW * 8, 128), jnp.float32),
            pltpu.VMEM((_BPW,), jnp.float32),
            pltpu.SemaphoreType.DMA,
        ],
        compiler_params=pltpu.CompilerParams(needs_layout_passes=False),
    )


def _sc_gather_call(logits, labels32):
    return _sc_gather()(logits, labels32)


_BM = 512
_BN = 4096


def _tc_blend_body(margin_ref, labels_ref, target_ref, logits_ref, out_ref, nv_ref):
    j = pl.program_id(1)

    @pl.when(j == 0)
    def _():
        m = margin_ref[...]
        t = target_ref[...]
        cos_m = jnp.cos(m)
        sin_m = jnp.sin(m)
        theta = jnp.cos(math.pi - m)
        sinmm = jnp.sin(math.pi - m) * m
        sin_t = jnp.sqrt(1.0 - t * t)
        cos_tm = t * cos_m - sin_t * sin_m
        nv_ref[...] = jnp.where(t > theta, cos_tm, t - sinmm) * SCALE

    x = logits_ref[...]
    rel = labels_ref[...] - j * _BN
    cols = lax.broadcasted_iota(jnp.int32, (_BM, _BN), 1)
    out_ref[...] = jnp.where(cols == rel, nv_ref[...], x * SCALE)


def _tc_blend(margin, labels, target, logits):
    grid = (B // _BM, pl.cdiv(C, _BN))
    row_spec = pl.BlockSpec((_BM, 1), lambda i, j: (i, 0))
    return pl.pallas_call(
        _tc_blend_body,
        grid=grid,
        in_specs=[
            row_spec,
            row_spec,
            row_spec,
            pl.BlockSpec((_BM, _BN), lambda i, j: (i, j)),
        ],
        out_specs=pl.BlockSpec((_BM, _BN), lambda i, j: (i, j)),
        out_shape=jax.ShapeDtypeStruct((B, C), jnp.float32),
        scratch_shapes=[pltpu.VMEM((_BM, 1), jnp.float32)],
    )(margin, labels, target, logits)


def kernel(margin, logits, labels):
    labels32 = labels.astype(jnp.int32)
    target = _sc_gather_call(logits, labels32)
    return _tc_blend(
        margin.reshape(B, 1), labels32.reshape(B, 1), target.reshape(B, 1), logits
    )
